# baseline (speedup 1.0000x reference)
import functools

import jax
import jax.numpy as jnp
from jax import lax
from jax.experimental import pallas as pl
from jax.experimental.pallas import tpu as pltpu
from jax.experimental.pallas import tpu_sc as plsc

N = 10000
E = 320000
CH = 128
NC = 2
NS = 16
NW = NC * NS
CHUNK = 128
CPW = 80
HCPW = CPW // 2
E_PAD = NW * CPW * CHUNK
RPS = 632
N_PAD = NS * RPS
DEG_PAD = 10240
DEG_STRIPE = DEG_PAD // NS

_mesh = plsc.VectorSubcoreMesh(core_axis_name="c", subcore_axis_name="s")


@functools.partial(
    pl.kernel,
    out_type=jax.ShapeDtypeStruct((NC * DEG_PAD,), jnp.float32),
    mesh=_mesh,
    scratch_types=[
        pltpu.VMEM((CPW, CHUNK), jnp.int32),
        pltpu.VMEM((CHUNK,), jnp.float32),
        pltpu.VMEM((DEG_STRIPE,), jnp.float32),
        pltpu.VMEM_SHARED((DEG_PAD,), jnp.float32),
    ],
)
def _deg(col2_hbm, out_hbm, cidx_all, ones_v, z_v, acc):
    c = lax.axis_index("c")
    s = lax.axis_index("s")
    wid = c * NS + s
    one16 = jnp.full((16,), 1.0, jnp.float32)
    zero16 = jnp.zeros((16,), jnp.float32)
    for j in range(CHUNK // 16):
        ones_v[pl.ds(j * 16, 16)] = one16
    for j in range(DEG_STRIPE // 16):
        z_v[pl.ds(j * 16, 16)] = zero16
    pltpu.sync_copy(z_v, acc.at[pl.ds(s * DEG_STRIPE, DEG_STRIPE)])
    pltpu.sync_copy(col2_hbm.at[pl.ds(wid * CPW, CPW)], cidx_all)
    plsc.subcore_barrier()

    for j in range(CPW):
        pltpu.sync_copy(ones_v, acc.at[cidx_all.at[j]], add=True)

    plsc.subcore_barrier()
    pltpu.sync_copy(
        acc.at[pl.ds(s * DEG_STRIPE, DEG_STRIPE)],
        out_hbm.at[pl.ds(c * DEG_PAD + s * DEG_STRIPE, DEG_STRIPE)],
    )


@functools.partial(
    pl.kernel,
    out_type=jax.ShapeDtypeStruct((NC * N_PAD, CH), jnp.float32),
    mesh=_mesh,
    scratch_types=[
        pltpu.VMEM((CPW, CHUNK), jnp.int32),
        pltpu.VMEM((CPW, CHUNK), jnp.int32),
        pltpu.VMEM((CHUNK, CH), jnp.float32),
        pltpu.VMEM_SHARED((N_PAD, CH), jnp.float32),
        pltpu.SemaphoreType.DMA,
    ],
)
def _prop(g_hbm, row2_hbm, col2_hbm, z_hbm, out_hbm,
          ridx_all, cidx_all, buf, acc, sem):
    c = lax.axis_index("c")
    s = lax.axis_index("s")
    wid = c * NS + s
    pltpu.sync_copy(z_hbm, acc.at[pl.ds(s * RPS, RPS)])
    pltpu.sync_copy(row2_hbm.at[pl.ds(wid * CPW, CPW)], ridx_all)
    pltpu.sync_copy(col2_hbm.at[pl.ds(wid * CPW, CPW)], cidx_all)
    plsc.subcore_barrier()

    for j in range(CPW):
        pltpu.async_copy(g_hbm.at[ridx_all.at[j]], buf, sem).wait()
        pltpu.sync_copy(buf, acc.at[cidx_all.at[j]], add=True)

    plsc.subcore_barrier()
    pltpu.sync_copy(
        acc.at[pl.ds(s * RPS, RPS)],
        out_hbm.at[pl.ds(c * N_PAD + s * RPS, RPS)],
    )


_BLK = 1000
_GRID = N // _BLK

_row_spec = pl.BlockSpec((_BLK, CH), lambda i: (i, 0))
_col_spec = pl.BlockSpec((_BLK, 1), lambda i: (i, 0))
_w_spec = pl.BlockSpec((CH, CH), lambda i: (0, 0))
_b_spec = pl.BlockSpec((1, CH), lambda i: (0, 0))


def _t0_body(x_r, w0_r, da_r, db_r, out_r, g_r, dinv_r):
    d = da_r[...] + db_r[...]
    dinv = jnp.where(d > 0, lax.rsqrt(jnp.maximum(d, 1e-12)), 0.0)
    x = x_r[...]
    out_r[...] = jnp.dot(x, w0_r[...], preferred_element_type=jnp.float32)
    g_r[...] = x * dinv
    dinv_r[...] = dinv


_t0 = pl.pallas_call(
    _t0_body,
    grid=(_GRID,),
    in_specs=[_row_spec, _w_spec, _col_spec, _col_spec],
    out_specs=[_row_spec, _row_spec, _col_spec],
    out_shape=[
        jax.ShapeDtypeStruct((N, CH), jnp.float32),
        jax.ShapeDtypeStruct((N, CH), jnp.float32),
        jax.ShapeDtypeStruct((N, 1), jnp.float32),
    ],
)


def _tmid_body(pa_r, pb_r, dinv_r, w_r, acc_r, out_r, g_r):
    dinv = dinv_r[...]
    q = (pa_r[...] + pb_r[...]) * dinv
    out_r[...] = acc_r[...] + jnp.dot(q, w_r[...], preferred_element_type=jnp.float32)
    g_r[...] = q * dinv


_tmid = pl.pallas_call(
    _tmid_body,
    grid=(_GRID,),
    in_specs=[_row_spec, _row_spec, _col_spec, _w_spec, _row_spec],
    out_specs=[_row_spec, _row_spec],
    out_shape=[
        jax.ShapeDtypeStruct((N, CH), jnp.float32),
        jax.ShapeDtypeStruct((N, CH), jnp.float32),
    ],
)


def _ttrans_body(pa_r, pb_r, dinv_r, w13_r, acc_r, b1_r, w20_r, b2_r, out_r, g_r):
    dinv = dinv_r[...]
    q = (pa_r[...] + pb_r[...]) * dinv
    h = acc_r[...] + jnp.dot(q, w13_r[...], preferred_element_type=jnp.float32) + b1_r[...]
    h = jnp.maximum(h, 0.0)
    out_r[...] = jnp.dot(h, w20_r[...], preferred_element_type=jnp.float32) + b2_r[...]
    g_r[...] = h * dinv


_ttrans = pl.pallas_call(
    _ttrans_body,
    grid=(_GRID,),
    in_specs=[_row_spec, _row_spec, _col_spec, _w_spec, _row_spec, _b_spec,
              _w_spec, _b_spec],
    out_specs=[_row_spec, _row_spec],
    out_shape=[
        jax.ShapeDtypeStruct((N, CH), jnp.float32),
        jax.ShapeDtypeStruct((N, CH), jnp.float32),
    ],
)


def _tfinal_body(pa_r, pb_r, dinv_r, w_r, acc_r, out_r):
    q = (pa_r[...] + pb_r[...]) * dinv_r[...]
    o = acc_r[...] + jnp.dot(q, w_r[...], preferred_element_type=jnp.float32)
    m = jnp.max(o, axis=1, keepdims=True)
    lse = jnp.log(jnp.sum(jnp.exp(o - m), axis=1, keepdims=True)) + m
    out_r[...] = o - lse


_tfinal = pl.pallas_call(
    _tfinal_body,
    grid=(_GRID,),
    in_specs=[_row_spec, _row_spec, _col_spec, _w_spec, _row_spec],
    out_specs=_row_spec,
    out_shape=jax.ShapeDtypeStruct((N, CH), jnp.float32),
)


def kernel(x, edge_index, W1_0, W1_1, W1_2, W1_3, b1, W2_0, W2_1, W2_2, W2_3, b2):
    pad = E_PAD - E
    row2 = jnp.concatenate(
        [edge_index[0].astype(jnp.int32), jnp.zeros((pad,), jnp.int32)]
    ).reshape(-1, CHUNK)
    col2 = jnp.concatenate(
        [edge_index[1].astype(jnp.int32), jnp.full((pad,), N, jnp.int32)]
    ).reshape(-1, CHUNK)
    zrows = jnp.zeros((RPS, CH), jnp.float32)
    b1r = b1.reshape(1, CH)
    b2r = b2.reshape(1, CH)

    degp = _deg(col2)
    da = degp[:N].reshape(N, 1)
    db = degp[DEG_PAD:DEG_PAD + N].reshape(N, 1)

    out, g, dinv = _t0(x, W1_0, da, db)

    for W in (W1_1, W1_2):
        p = _prop(g, row2, col2, zrows)
        out, g = _tmid(p[:N], p[N_PAD:N_PAD + N], dinv, W, out)

    p = _prop(g, row2, col2, zrows)
    out, g = _ttrans(p[:N], p[N_PAD:N_PAD + N], dinv, W1_3, out, b1r, W2_0, b2r)

    for W in (W2_1, W2_2):
        p = _prop(g, row2, col2, zrows)
        out, g = _tmid(p[:N], p[N_PAD:N_PAD + N], dinv, W, out)

    p = _prop(g, row2, col2, zrows)
    return _tfinal(p[:N], p[N_PAD:N_PAD + N], dinv, W2_3, out)


# speedup vs baseline: 2.7527x; 2.7527x over previous
import functools

import jax
import jax.numpy as jnp
from jax import lax
from jax.experimental import pallas as pl
from jax.experimental.pallas import tpu as pltpu
from jax.experimental.pallas import tpu_sc as plsc

N = 10000
E = 320000
CH = 128
NC = 2
NS = 16
NW = NC * NS
CHUNK = 128
CPW = 80
HCPW = CPW // 2
E_PAD = NW * CPW * CHUNK
RPS = 632
N_PAD = NS * RPS
DEG_PAD = 10240
DEG_STRIPE = DEG_PAD // NS

_mesh = plsc.VectorSubcoreMesh(core_axis_name="c", subcore_axis_name="s")


@functools.partial(
    pl.kernel,
    out_type=jax.ShapeDtypeStruct((NC * DEG_PAD,), jnp.float32),
    mesh=_mesh,
    scratch_types=[
        pltpu.VMEM((CPW, CHUNK), jnp.int32),
        pltpu.VMEM((CHUNK,), jnp.float32),
        pltpu.VMEM((DEG_STRIPE,), jnp.float32),
        pltpu.VMEM_SHARED((DEG_PAD,), jnp.float32),
    ],
)
def _deg(col2_hbm, out_hbm, cidx_all, ones_v, z_v, acc):
    c = lax.axis_index("c")
    s = lax.axis_index("s")
    wid = c * NS + s
    one16 = jnp.full((16,), 1.0, jnp.float32)
    zero16 = jnp.zeros((16,), jnp.float32)
    for j in range(CHUNK // 16):
        ones_v[pl.ds(j * 16, 16)] = one16
    for j in range(DEG_STRIPE // 16):
        z_v[pl.ds(j * 16, 16)] = zero16
    pltpu.sync_copy(z_v, acc.at[pl.ds(s * DEG_STRIPE, DEG_STRIPE)])
    pltpu.sync_copy(col2_hbm.at[pl.ds(wid * CPW, CPW)], cidx_all)
    plsc.subcore_barrier()

    for j in range(CPW):
        pltpu.sync_copy(ones_v, acc.at[cidx_all.at[j]], add=True)

    plsc.subcore_barrier()
    pltpu.sync_copy(
        acc.at[pl.ds(s * DEG_STRIPE, DEG_STRIPE)],
        out_hbm.at[pl.ds(c * DEG_PAD + s * DEG_STRIPE, DEG_STRIPE)],
    )


@functools.partial(
    pl.kernel,
    out_type=jax.ShapeDtypeStruct((NC * N_PAD, CH), jnp.float32),
    mesh=_mesh,
    scratch_types=[
        pltpu.VMEM((CPW, CHUNK), jnp.int32),
        pltpu.VMEM((CPW, CHUNK), jnp.int32),
        pltpu.VMEM((CHUNK, CH), jnp.float32),
        pltpu.VMEM_SHARED((N_PAD, CH), jnp.float32),
        pltpu.SemaphoreType.DMA,
    ],
)
def _prop(g_hbm, row2_hbm, col2_hbm, z_hbm, out_hbm,
          ridx_all, cidx_all, buf, acc, sem):
    c = lax.axis_index("c")
    s = lax.axis_index("s")
    wid = c * NS + s
    pltpu.sync_copy(z_hbm, acc.at[pl.ds(s * RPS, RPS)])
    pltpu.sync_copy(row2_hbm.at[pl.ds(wid * CPW, CPW)], ridx_all)
    pltpu.sync_copy(col2_hbm.at[pl.ds(wid * CPW, CPW)], cidx_all)
    plsc.subcore_barrier()

    for j in range(CPW):
        pltpu.async_copy(g_hbm.at[ridx_all.at[j]], buf, sem).wait()
        pltpu.sync_copy(buf, acc.at[cidx_all.at[j]], add=True)

    plsc.subcore_barrier()
    pltpu.sync_copy(
        acc.at[pl.ds(s * RPS, RPS)],
        out_hbm.at[pl.ds(c * N_PAD + s * RPS, RPS)],
    )


_BLK = 1000
_GRID = N // _BLK

_row_spec = pl.BlockSpec((_BLK, CH), lambda i: (i, 0))
_col_spec = pl.BlockSpec((_BLK, 1), lambda i: (i, 0))
_w_spec = pl.BlockSpec((CH, CH), lambda i: (0, 0))
_b_spec = pl.BlockSpec((1, CH), lambda i: (0, 0))


def _t0_body(x_r, w0_r, da_r, db_r, out_r, g_r, dinv_r):
    d = da_r[...] + db_r[...]
    dinv = jnp.where(d > 0, lax.rsqrt(jnp.maximum(d, 1e-12)), 0.0)
    x = x_r[...]
    out_r[...] = jnp.dot(x, w0_r[...], preferred_element_type=jnp.float32)
    g_r[...] = x * dinv
    dinv_r[...] = dinv


_t0 = pl.pallas_call(
    _t0_body,
    grid=(_GRID,),
    in_specs=[_row_spec, _w_spec, _col_spec, _col_spec],
    out_specs=[_row_spec, _row_spec, _col_spec],
    out_shape=[
        jax.ShapeDtypeStruct((N, CH), jnp.float32),
        jax.ShapeDtypeStruct((N, CH), jnp.float32),
        jax.ShapeDtypeStruct((N, 1), jnp.float32),
    ],
)


def _tmid_body(pa_r, pb_r, dinv_r, w_r, acc_r, out_r, g_r):
    dinv = dinv_r[...]
    q = (pa_r[...] + pb_r[...]) * dinv
    out_r[...] = acc_r[...] + jnp.dot(q, w_r[...], preferred_element_type=jnp.float32)
    g_r[...] = q * dinv


_tmid = pl.pallas_call(
    _tmid_body,
    grid=(_GRID,),
    in_specs=[_row_spec, _row_spec, _col_spec, _w_spec, _row_spec],
    out_specs=[_row_spec, _row_spec],
    out_shape=[
        jax.ShapeDtypeStruct((N, CH), jnp.float32),
        jax.ShapeDtypeStruct((N, CH), jnp.float32),
    ],
)


def _ttrans_body(pa_r, pb_r, dinv_r, w13_r, acc_r, b1_r, w20_r, b2_r, out_r, g_r):
    dinv = dinv_r[...]
    q = (pa_r[...] + pb_r[...]) * dinv
    h = acc_r[...] + jnp.dot(q, w13_r[...], preferred_element_type=jnp.float32) + b1_r[...]
    h = jnp.maximum(h, 0.0)
    out_r[...] = jnp.dot(h, w20_r[...], preferred_element_type=jnp.float32) + b2_r[...]
    g_r[...] = h * dinv


_ttrans = pl.pallas_call(
    _ttrans_body,
    grid=(_GRID,),
    in_specs=[_row_spec, _row_spec, _col_spec, _w_spec, _row_spec, _b_spec,
              _w_spec, _b_spec],
    out_specs=[_row_spec, _row_spec],
    out_shape=[
        jax.ShapeDtypeStruct((N, CH), jnp.float32),
        jax.ShapeDtypeStruct((N, CH), jnp.float32),
    ],
)


def _tfinal_body(pa_r, pb_r, dinv_r, w_r, acc_r, out_r):
    q = (pa_r[...] + pb_r[...]) * dinv_r[...]
    o = acc_r[...] + jnp.dot(q, w_r[...], preferred_element_type=jnp.float32)
    m = jnp.max(o, axis=1, keepdims=True)
    lse = jnp.log(jnp.sum(jnp.exp(o - m), axis=1, keepdims=True)) + m
    out_r[...] = o - lse


_tfinal = pl.pallas_call(
    _tfinal_body,
    grid=(_GRID,),
    in_specs=[_row_spec, _row_spec, _col_spec, _w_spec, _row_spec],
    out_specs=_row_spec,
    out_shape=jax.ShapeDtypeStruct((N, CH), jnp.float32),
)


def kernel(x, edge_index, W1_0, W1_1, W1_2, W1_3, b1, W2_0, W2_1, W2_2, W2_3, b2):
    pad = E_PAD - E
    parange = jnp.arange(pad, dtype=jnp.int32)
    row2 = jnp.concatenate(
        [edge_index[0].astype(jnp.int32), parange % N]
    ).reshape(-1, CHUNK)
    col2 = jnp.concatenate(
        [edge_index[1].astype(jnp.int32), N + parange % (N_PAD - N)]
    ).reshape(-1, CHUNK)
    zrows = jnp.zeros((RPS, CH), jnp.float32)
    b1r = b1.reshape(1, CH)
    b2r = b2.reshape(1, CH)

    degp = _deg(col2)
    da = degp[:N].reshape(N, 1)
    db = degp[DEG_PAD:DEG_PAD + N].reshape(N, 1)

    out, g, dinv = _t0(x, W1_0, da, db)

    for W in (W1_1, W1_2):
        p = _prop(g, row2, col2, zrows)
        out, g = _tmid(p[:N], p[N_PAD:N_PAD + N], dinv, W, out)

    p = _prop(g, row2, col2, zrows)
    out, g = _ttrans(p[:N], p[N_PAD:N_PAD + N], dinv, W1_3, out, b1r, W2_0, b2r)

    for W in (W2_1, W2_2):
        p = _prop(g, row2, col2, zrows)
        out, g = _tmid(p[:N], p[N_PAD:N_PAD + N], dinv, W, out)

    p = _prop(g, row2, col2, zrows)
    return _tfinal(p[:N], p[N_PAD:N_PAD + N], dinv, W2_3, out)


# speedup vs baseline: 4.0376x; 1.4668x over previous
import functools

import jax
import jax.numpy as jnp
from jax import lax
from jax.experimental import pallas as pl
from jax.experimental.pallas import tpu as pltpu
from jax.experimental.pallas import tpu_sc as plsc

N = 10000
E = 320000
CH = 128
NC = 2
NS = 16
NW = NC * NS
CHUNK = 128
CPW = 80
HCPW = CPW // 2
E_PAD = NW * CPW * CHUNK
RPS = 632
N_PAD = NS * RPS
DEG_PAD = 10240
DEG_STRIPE = DEG_PAD // NS

_mesh = plsc.VectorSubcoreMesh(core_axis_name="c", subcore_axis_name="s")


@functools.partial(
    pl.kernel,
    out_type=jax.ShapeDtypeStruct((NC * DEG_PAD,), jnp.float32),
    mesh=_mesh,
    scratch_types=[
        pltpu.VMEM((CPW, CHUNK), jnp.int32),
        pltpu.VMEM((CHUNK,), jnp.float32),
        pltpu.VMEM((DEG_STRIPE,), jnp.float32),
        pltpu.VMEM_SHARED((DEG_PAD,), jnp.float32),
    ],
)
def _deg(col2_hbm, out_hbm, cidx_all, ones_v, z_v, acc):
    c = lax.axis_index("c")
    s = lax.axis_index("s")
    wid = c * NS + s
    one16 = jnp.full((16,), 1.0, jnp.float32)
    zero16 = jnp.zeros((16,), jnp.float32)
    for j in range(CHUNK // 16):
        ones_v[pl.ds(j * 16, 16)] = one16
    for j in range(DEG_STRIPE // 16):
        z_v[pl.ds(j * 16, 16)] = zero16
    pltpu.sync_copy(z_v, acc.at[pl.ds(s * DEG_STRIPE, DEG_STRIPE)])
    pltpu.sync_copy(col2_hbm.at[pl.ds(wid * CPW, CPW)], cidx_all)
    plsc.subcore_barrier()

    for j in range(CPW):
        pltpu.sync_copy(ones_v, acc.at[cidx_all.at[j]], add=True)

    plsc.subcore_barrier()
    pltpu.sync_copy(
        acc.at[pl.ds(s * DEG_STRIPE, DEG_STRIPE)],
        out_hbm.at[pl.ds(c * DEG_PAD + s * DEG_STRIPE, DEG_STRIPE)],
    )


@functools.partial(
    pl.kernel,
    out_type=jax.ShapeDtypeStruct((NC * N_PAD, CH), jnp.float32),
    mesh=_mesh,
    scratch_types=[
        pltpu.VMEM((HCPW, CHUNK), jnp.int32),
        pltpu.VMEM((HCPW, CHUNK), jnp.int32),
        pltpu.VMEM((CHUNK, CH), jnp.float32),
        pltpu.VMEM((CHUNK, CH), jnp.float32),
        pltpu.VMEM_SHARED((N_PAD, CH), jnp.float32),
        pltpu.SemaphoreType.DMA,
        pltpu.SemaphoreType.DMA,
    ],
)
def _prop(g_hbm, row2_hbm, col2_hbm, z_hbm, out_hbm,
          ridx_all, cidx_all, bufa, bufb, acc, sema, semb):
    c = lax.axis_index("c")
    s = lax.axis_index("s")
    wid = c * NS + s
    pltpu.sync_copy(z_hbm, acc.at[pl.ds(s * RPS, RPS)])
    plsc.subcore_barrier()

    bufs = (bufa, bufb)
    sems = (sema, semb)
    for ph in range(2):
        base = wid * CPW + ph * HCPW
        pltpu.sync_copy(row2_hbm.at[pl.ds(base, HCPW)], ridx_all)
        pltpu.sync_copy(col2_hbm.at[pl.ds(base, HCPW)], cidx_all)
        descs = [None, None]
        descs[0] = pltpu.async_copy(g_hbm.at[ridx_all.at[0]], bufa, sema)
        for j in range(HCPW):
            p = j % 2
            if j + 1 < HCPW:
                descs[1 - p] = pltpu.async_copy(
                    g_hbm.at[ridx_all.at[j + 1]], bufs[1 - p], sems[1 - p])
            descs[p].wait()
            pltpu.sync_copy(bufs[p], acc.at[cidx_all.at[j]], add=True)

    plsc.subcore_barrier()
    pltpu.sync_copy(
        acc.at[pl.ds(s * RPS, RPS)],
        out_hbm.at[pl.ds(c * N_PAD + s * RPS, RPS)],
    )


_BLK = 1000
_GRID = N // _BLK

_row_spec = pl.BlockSpec((_BLK, CH), lambda i: (i, 0))
_col_spec = pl.BlockSpec((_BLK, 1), lambda i: (i, 0))
_w_spec = pl.BlockSpec((CH, CH), lambda i: (0, 0))
_b_spec = pl.BlockSpec((1, CH), lambda i: (0, 0))


def _t0_body(x_r, w0_r, da_r, db_r, out_r, g_r, dinv_r):
    d = da_r[...] + db_r[...]
    dinv = jnp.where(d > 0, lax.rsqrt(jnp.maximum(d, 1e-12)), 0.0)
    x = x_r[...]
    out_r[...] = jnp.dot(x, w0_r[...], preferred_element_type=jnp.float32)
    g_r[...] = x * dinv
    dinv_r[...] = dinv


_t0 = pl.pallas_call(
    _t0_body,
    grid=(_GRID,),
    in_specs=[_row_spec, _w_spec, _col_spec, _col_spec],
    out_specs=[_row_spec, _row_spec, _col_spec],
    out_shape=[
        jax.ShapeDtypeStruct((N, CH), jnp.float32),
        jax.ShapeDtypeStruct((N, CH), jnp.float32),
        jax.ShapeDtypeStruct((N, 1), jnp.float32),
    ],
)


def _tmid_body(pa_r, pb_r, dinv_r, w_r, acc_r, out_r, g_r):
    dinv = dinv_r[...]
    q = (pa_r[...] + pb_r[...]) * dinv
    out_r[...] = acc_r[...] + jnp.dot(q, w_r[...], preferred_element_type=jnp.float32)
    g_r[...] = q * dinv


_tmid = pl.pallas_call(
    _tmid_body,
    grid=(_GRID,),
    in_specs=[_row_spec, _row_spec, _col_spec, _w_spec, _row_spec],
    out_specs=[_row_spec, _row_spec],
    out_shape=[
        jax.ShapeDtypeStruct((N, CH), jnp.float32),
        jax.ShapeDtypeStruct((N, CH), jnp.float32),
    ],
)


def _ttrans_body(pa_r, pb_r, dinv_r, w13_r, acc_r, b1_r, w20_r, b2_r, out_r, g_r):
    dinv = dinv_r[...]
    q = (pa_r[...] + pb_r[...]) * dinv
    h = acc_r[...] + jnp.dot(q, w13_r[...], preferred_element_type=jnp.float32) + b1_r[...]
    h = jnp.maximum(h, 0.0)
    out_r[...] = jnp.dot(h, w20_r[...], preferred_element_type=jnp.float32) + b2_r[...]
    g_r[...] = h * dinv


_ttrans = pl.pallas_call(
    _ttrans_body,
    grid=(_GRID,),
    in_specs=[_row_spec, _row_spec, _col_spec, _w_spec, _row_spec, _b_spec,
              _w_spec, _b_spec],
    out_specs=[_row_spec, _row_spec],
    out_shape=[
        jax.ShapeDtypeStruct((N, CH), jnp.float32),
        jax.ShapeDtypeStruct((N, CH), jnp.float32),
    ],
)


def _tfinal_body(pa_r, pb_r, dinv_r, w_r, acc_r, out_r):
    q = (pa_r[...] + pb_r[...]) * dinv_r[...]
    o = acc_r[...] + jnp.dot(q, w_r[...], preferred_element_type=jnp.float32)
    m = jnp.max(o, axis=1, keepdims=True)
    lse = jnp.log(jnp.sum(jnp.exp(o - m), axis=1, keepdims=True)) + m
    out_r[...] = o - lse


_tfinal = pl.pallas_call(
    _tfinal_body,
    grid=(_GRID,),
    in_specs=[_row_spec, _row_spec, _col_spec, _w_spec, _row_spec],
    out_specs=_row_spec,
    out_shape=jax.ShapeDtypeStruct((N, CH), jnp.float32),
)


def kernel(x, edge_index, W1_0, W1_1, W1_2, W1_3, b1, W2_0, W2_1, W2_2, W2_3, b2):
    pad = E_PAD - E
    parange = jnp.arange(pad, dtype=jnp.int32)
    row2 = jnp.concatenate(
        [edge_index[0].astype(jnp.int32), parange % N]
    ).reshape(-1, CHUNK)
    col2 = jnp.concatenate(
        [edge_index[1].astype(jnp.int32), N + parange % (N_PAD - N)]
    ).reshape(-1, CHUNK)
    zrows = jnp.zeros((RPS, CH), jnp.float32)
    b1r = b1.reshape(1, CH)
    b2r = b2.reshape(1, CH)

    degp = _deg(col2)
    da = degp[:N].reshape(N, 1)
    db = degp[DEG_PAD:DEG_PAD + N].reshape(N, 1)

    out, g, dinv = _t0(x, W1_0, da, db)

    for W in (W1_1, W1_2):
        p = _prop(g, row2, col2, zrows)
        out, g = _tmid(p[:N], p[N_PAD:N_PAD + N], dinv, W, out)

    p = _prop(g, row2, col2, zrows)
    out, g = _ttrans(p[:N], p[N_PAD:N_PAD + N], dinv, W1_3, out, b1r, W2_0, b2r)

    for W in (W2_1, W2_2):
        p = _prop(g, row2, col2, zrows)
        out, g = _tmid(p[:N], p[N_PAD:N_PAD + N], dinv, W, out)

    p = _prop(g, row2, col2, zrows)
    return _tfinal(p[:N], p[N_PAD:N_PAD + N], dinv, W2_3, out)
